# baseline (speedup 1.0000x reference)
import jax
import jax.numpy as jnp
from jax.experimental import pallas as pl
from jax.experimental.pallas import tpu as pltpu

SMOOTHING_VALUE = 0.1
PAD_TOKEN_ID = 0
TRG_VOCAB_SIZE = 100000
CONFIDENCE_VALUE = 1.0 - SMOOTHING_VALUE
SMOOTH = SMOOTHING_VALUE / (TRG_VOCAB_SIZE - 2)

ROWS = 8
NBUF = 8
BATCH = 1024
NSTEPS = BATCH // ROWS


def _fill_kernel(idx_ref, out_ref, buf, sem):
    g = pl.program_id(0)
    slot = jax.lax.rem(g, NBUF)

    @pl.when(g >= NBUF)
    def _():
        pltpu.make_async_copy(
            buf.at[slot], out_ref.at[pl.ds((g - NBUF) * ROWS, ROWS), :], sem.at[slot]
        ).wait()

    idx = idx_ref[pl.ds(g * ROWS, ROWS), :]
    cols = jax.lax.broadcasted_iota(jnp.int32, (ROWS, TRG_VOCAB_SIZE), 1)
    val = jnp.where(cols == idx, jnp.float32(CONFIDENCE_VALUE), jnp.float32(SMOOTH))
    val = jnp.where(idx == PAD_TOKEN_ID, jnp.float32(0.0), val)
    buf[slot] = val

    pltpu.make_async_copy(
        buf.at[slot], out_ref.at[pl.ds(g * ROWS, ROWS), :], sem.at[slot]
    ).start()

    @pl.when(g == NSTEPS - 1)
    def _():
        for k in range(NBUF):
            step = NSTEPS - NBUF + k
            s = step % NBUF
            pltpu.make_async_copy(
                buf.at[s], out_ref.at[pl.ds(step * ROWS, ROWS), :], sem.at[s]
            ).wait()


@jax.jit
def kernel(trg_token_ids_batch):
    idx = trg_token_ids_batch.astype(jnp.int32)
    return pl.pallas_call(
        _fill_kernel,
        grid=(NSTEPS,),
        in_specs=[pl.BlockSpec(memory_space=pltpu.MemorySpace.VMEM)],
        out_specs=pl.BlockSpec(memory_space=pltpu.MemorySpace.HBM),
        out_shape=jax.ShapeDtypeStruct((BATCH, TRG_VOCAB_SIZE), jnp.float32),
        scratch_shapes=[
            pltpu.VMEM((NBUF, ROWS, TRG_VOCAB_SIZE), jnp.float32),
            pltpu.SemaphoreType.DMA((NBUF,)),
        ],
    )(idx)
